# Initial kernel scaffold; baseline (speedup 1.0000x reference)
import jax
import jax.numpy as jnp
from jax.experimental import pallas as pl


def kernel(feat, edge_index, edge_order, o_emb_weight, bias):
    raise NotImplementedError("write your pallas kernel here")


# speedup vs baseline: 10.6862x; 10.6862x over previous
import functools

import jax
import jax.numpy as jnp
from jax import lax
from jax.experimental import pallas as pl
from jax.experimental.pallas import tpu as pltpu
from jax.experimental.pallas import tpu_sc as plsc

N = 10000
NP = 10240
E = 320000
D = 128
NORD = 10
NC, NS, L = 2, 16, 16
NW = NC * NS

ROWS_PER_TILE = NP // NS
EDGES_PER_TILE_A = E // NS
EDGES_PER_W = E // NW
K = 80


def _zero_rows(zbuf, nrows, ncols):
    zeros16 = jnp.zeros((L,), jnp.float32)

    def body(i, _):
        for j in range(ncols // L):
            zbuf[i, pl.ds(j * L, L)] = zeros16
        return 0

    lax.fori_loop(0, nrows, body, 0)


def _count_loop(idx_hbm, acc, ones_b, idx_b, out_ref, s):
    base = s * EDGES_PER_TILE_A

    def body(i, _):
        pltpu.sync_copy(idx_hbm.at[pl.ds(base + i * K, K)], idx_b)
        pltpu.sync_copy(ones_b, acc.at[idx_b], add=True)
        return 0

    lax.fori_loop(0, EDGES_PER_TILE_A // K, body, 0)
    plsc.subcore_barrier()
    row0 = s * ROWS_PER_TILE
    pltpu.sync_copy(acc.at[pl.ds(row0, ROWS_PER_TILE)],
                    out_ref.at[pl.ds(row0, ROWS_PER_TILE)])


def _degrees(src, dst):
    mesh = plsc.VectorSubcoreMesh(core_axis_name="c", subcore_axis_name="s")

    @functools.partial(
        pl.kernel,
        mesh=mesh,
        out_type=(
            jax.ShapeDtypeStruct((NP, L), jnp.float32),
            jax.ShapeDtypeStruct((NP, L), jnp.float32),
        ),
        scratch_types=[
            pltpu.VMEM_SHARED((NP, L), jnp.float32),
            pltpu.VMEM((ROWS_PER_TILE, L), jnp.float32),
            pltpu.VMEM((K, L), jnp.float32),
            pltpu.VMEM((K,), jnp.int32),
        ],
        compiler_params=pltpu.CompilerParams(use_tc_tiling_on_sc=False),
    )
    def k(src_hbm, dst_hbm, ocnt_hbm, icnt_hbm, acc, zbuf, ones_b, idx_b):
        c = lax.axis_index("c")
        s = lax.axis_index("s")
        _zero_rows(zbuf, ROWS_PER_TILE, L)
        pltpu.sync_copy(zbuf, acc.at[pl.ds(s * ROWS_PER_TILE, ROWS_PER_TILE)])
        ones16 = jnp.ones((L,), jnp.float32)
        for i in range(K):
            ones_b[i, :] = ones16
        plsc.subcore_barrier()

        @pl.when(c == 0)
        def _():
            _count_loop(src_hbm, acc, ones_b, idx_b, ocnt_hbm, s)

        @pl.when(c == 1)
        def _():
            _count_loop(dst_hbm, acc, ones_b, idx_b, icnt_hbm, s)

    return k(src, dst)


def _transform_body(feat_ref, cnt_ref, wt_ref, out_ref):
    cnt = cnt_ref[:, 0:1]
    scale = 1.0 / jnp.sqrt(jnp.maximum(cnt, 1.0))
    out_ref[...] = jnp.dot(feat_ref[...] * scale, wt_ref[...],
                           preferred_element_type=jnp.float32)


def _transform(feat, ocnt, wt):
    blk = 1000
    return pl.pallas_call(
        _transform_body,
        grid=(N // blk,),
        in_specs=[
            pl.BlockSpec((blk, D), lambda i: (i, 0)),
            pl.BlockSpec((blk, L), lambda i: (i, 0)),
            pl.BlockSpec((D, NORD * D), lambda i: (0, 0)),
        ],
        out_specs=pl.BlockSpec((blk, NORD * D), lambda i: (i, 0)),
        out_shape=jax.ShapeDtypeStruct((N, NORD * D), jnp.float32),
    )(feat, ocnt, wt)


def _gather_scatter(tt, src, dst, order):
    mesh = plsc.VectorSubcoreMesh(core_axis_name="c", subcore_axis_name="s")

    @functools.partial(
        pl.kernel,
        mesh=mesh,
        out_type=jax.ShapeDtypeStruct((NC, NP, D), jnp.float32),
        scratch_types=[
            pltpu.VMEM_SHARED((NP, D), jnp.float32),
            pltpu.VMEM((ROWS_PER_TILE // 5, D), jnp.float32),
            pltpu.VMEM((K,), jnp.int32),
            pltpu.VMEM((K,), jnp.int32),
            pltpu.VMEM((K,), jnp.int32),
            pltpu.VMEM((K, D), jnp.float32),
            pltpu.SemaphoreType.DMA,
        ],
    )
    def k(tt_hbm, src_hbm, dst_hbm, ord_hbm, part_hbm,
          acc, zbuf, src_b, dst_b, gidx_b, rows, sem):
        c = lax.axis_index("c")
        s = lax.axis_index("s")
        wid = s * NC + c
        row0 = s * ROWS_PER_TILE

        _zero_rows(zbuf, ROWS_PER_TILE // 5, D)
        for r in range(5):
            pltpu.sync_copy(
                zbuf, acc.at[pl.ds(row0 + r * (ROWS_PER_TILE // 5),
                                   ROWS_PER_TILE // 5)])
        plsc.subcore_barrier()

        base = wid * EDGES_PER_W

        def body(i, _):
            e0 = base + i * K
            pltpu.sync_copy(src_hbm.at[pl.ds(e0, K)], src_b)
            pltpu.sync_copy(ord_hbm.at[pl.ds(e0, K)], gidx_b)
            pltpu.sync_copy(dst_hbm.at[pl.ds(e0, K)], dst_b)
            for j in range(K // L):
                sl = pl.ds(j * L, L)
                gidx_b[sl] = src_b[sl] * NORD + gidx_b[sl]
            pltpu.async_copy(tt_hbm.at[gidx_b], rows, sem).wait()
            pltpu.sync_copy(rows, acc.at[dst_b], add=True)
            return 0

        lax.fori_loop(0, EDGES_PER_W // K, body, 0)
        plsc.subcore_barrier()

        @pl.when(c == 0)
        def _():
            pltpu.sync_copy(acc.at[pl.ds(row0, ROWS_PER_TILE)],
                            part_hbm.at[0, pl.ds(row0, ROWS_PER_TILE)])

        @pl.when(c == 1)
        def _():
            pltpu.sync_copy(acc.at[pl.ds(row0, ROWS_PER_TILE)],
                            part_hbm.at[1, pl.ds(row0, ROWS_PER_TILE)])

    return k(tt, src, dst, order)


def _final_body(part_ref, cnt_ref, bias_ref, out_ref):
    p = part_ref[0] + part_ref[1]
    cnt = cnt_ref[:, 0:1]
    scale = 1.0 / jnp.sqrt(jnp.maximum(cnt, 1.0))
    out_ref[...] = p * scale + bias_ref[...]


def _finalize(part, icnt, bias2d):
    blk = 1000
    return pl.pallas_call(
        _final_body,
        grid=(N // blk,),
        in_specs=[
            pl.BlockSpec((NC, blk, D), lambda i: (0, i, 0)),
            pl.BlockSpec((blk, L), lambda i: (i, 0)),
            pl.BlockSpec((1, D), lambda i: (0, 0)),
        ],
        out_specs=pl.BlockSpec((blk, D), lambda i: (i, 0)),
        out_shape=jax.ShapeDtypeStruct((N, D), jnp.float32),
    )(part, icnt, bias2d)


def kernel(feat, edge_index, edge_order, o_emb_weight, bias):
    ei = edge_index.astype(jnp.int32)
    src = ei[0]
    dst = ei[1]
    order = edge_order.astype(jnp.int32)
    ocnt, icnt = _degrees(src, dst)
    wt = o_emb_weight.reshape(NORD, D, D).transpose(2, 0, 1).reshape(D, NORD * D)
    t = _transform(feat, ocnt, wt)
    tt = t.reshape(N * NORD, D)
    part = _gather_scatter(tt, src, dst, order)
    return _finalize(part, icnt, bias.reshape(1, D))


# speedup vs baseline: 21.9670x; 2.0556x over previous
import functools

import jax
import jax.numpy as jnp
from jax import lax
from jax.experimental import pallas as pl
from jax.experimental.pallas import tpu as pltpu
from jax.experimental.pallas import tpu_sc as plsc

N = 10000
NP = 10240
E = 320000
D = 128
NORD = 10
NC, NS, L = 2, 16, 16
NW = NC * NS

ROWS_PER_TILE = NP // NS
K = 80
NCH_A = E // NS // K
NCH_C = E // NW // K


def _zero_rows(zbuf, nrows, ncols):
    zeros16 = jnp.zeros((L,), jnp.float32)

    def body(i, _):
        for j in range(ncols // L):
            zbuf[i, pl.ds(j * L, L)] = zeros16
        return 0

    lax.fori_loop(0, nrows, body, 0)


def _degrees(src_a, dst_a):
    mesh = plsc.VectorSubcoreMesh(core_axis_name="c", subcore_axis_name="s")

    @functools.partial(
        pl.kernel,
        mesh=mesh,
        out_type=(
            jax.ShapeDtypeStruct((NP, L), jnp.float32),
            jax.ShapeDtypeStruct((NP, L), jnp.float32),
        ),
        scratch_types=[
            pltpu.VMEM_SHARED((NP, L), jnp.float32),
            pltpu.VMEM((ROWS_PER_TILE, L), jnp.float32),
            pltpu.VMEM((K, L), jnp.float32),
            pltpu.VMEM((NCH_A, K), jnp.int32),
            pltpu.SemaphoreType.DMA,
        ],
        compiler_params=pltpu.CompilerParams(use_tc_tiling_on_sc=False),
    )
    def k(src_hbm, dst_hbm, ocnt_hbm, icnt_hbm, acc, zbuf, ones_b, idx_all,
          sem):
        c = lax.axis_index("c")
        s = lax.axis_index("s")

        @pl.when(c == 0)
        def _():
            pltpu.sync_copy(src_hbm.at[s], idx_all)

        @pl.when(c == 1)
        def _():
            pltpu.sync_copy(dst_hbm.at[s], idx_all)

        _zero_rows(zbuf, ROWS_PER_TILE, L)
        pltpu.sync_copy(zbuf, acc.at[pl.ds(s * ROWS_PER_TILE, ROWS_PER_TILE)])
        ones16 = jnp.ones((L,), jnp.float32)
        for i in range(K):
            ones_b[i, :] = ones16
        plsc.subcore_barrier()

        def gbody(g, _):
            for u in range(5):
                pltpu.async_copy(ones_b, acc.at[idx_all.at[g * 5 + u]], sem,
                                 add=True)
            for u in range(5):
                pltpu.make_async_copy(ones_b, acc.at[idx_all.at[0]],
                                      sem).wait()
            return 0

        lax.fori_loop(0, NCH_A // 5, gbody, 0)
        plsc.subcore_barrier()
        row0 = s * ROWS_PER_TILE

        @pl.when(c == 0)
        def _():
            pltpu.sync_copy(acc.at[pl.ds(row0, ROWS_PER_TILE)],
                            ocnt_hbm.at[pl.ds(row0, ROWS_PER_TILE)])

        @pl.when(c == 1)
        def _():
            pltpu.sync_copy(acc.at[pl.ds(row0, ROWS_PER_TILE)],
                            icnt_hbm.at[pl.ds(row0, ROWS_PER_TILE)])

    return k(src_a, dst_a)


def _transform_body(feat_ref, cnt_ref, wt_ref, out_ref):
    cnt = cnt_ref[:, 0:1]
    scale = 1.0 / jnp.sqrt(jnp.maximum(cnt, 1.0))
    out_ref[...] = jnp.dot(feat_ref[...] * scale, wt_ref[...],
                           preferred_element_type=jnp.float32)


def _transform(feat, ocnt, wt):
    blk = 1000
    return pl.pallas_call(
        _transform_body,
        grid=(N // blk,),
        in_specs=[
            pl.BlockSpec((blk, D), lambda i: (i, 0)),
            pl.BlockSpec((blk, L), lambda i: (i, 0)),
            pl.BlockSpec((D, NORD * D), lambda i: (0, 0)),
        ],
        out_specs=pl.BlockSpec((blk, NORD * D), lambda i: (i, 0)),
        out_shape=jax.ShapeDtypeStruct((N, NORD * D), jnp.float32),
    )(feat, ocnt, wt)


def _gather_scatter(tt, gidx_c, dst_c):
    mesh = plsc.VectorSubcoreMesh(core_axis_name="c", subcore_axis_name="s")
    SEC, CPS = 5, NCH_C // 5

    @functools.partial(
        pl.kernel,
        mesh=mesh,
        out_type=jax.ShapeDtypeStruct((NC, NP, D), jnp.float32),
        scratch_types=[
            pltpu.VMEM_SHARED((NP, D), jnp.float32),
            pltpu.VMEM((CPS, K), jnp.int32),
            pltpu.VMEM((CPS, K), jnp.int32),
            pltpu.VMEM((K, D), jnp.float32),
            pltpu.VMEM((K, D), jnp.float32),
            pltpu.SemaphoreType.DMA,
            pltpu.SemaphoreType.DMA,
        ],
        compiler_params=pltpu.CompilerParams(use_tc_tiling_on_sc=False),
    )
    def k(tt_hbm, gidx_hbm, dst_hbm, part_hbm,
          acc, gidx_sec, dst_sec, rows0, rows1, sem0, sem1):
        c = lax.axis_index("c")
        s = lax.axis_index("s")
        wid = s * NC + c
        row0 = s * ROWS_PER_TILE

        _zero_rows(rows0, K, D)
        for r in range(8):
            pltpu.sync_copy(rows0, acc.at[pl.ds(row0 + r * K, K)])
        plsc.subcore_barrier()

        def sbody(sec, _):
            pltpu.sync_copy(gidx_hbm.at[wid, pl.ds(sec * CPS, CPS)], gidx_sec)
            pltpu.sync_copy(dst_hbm.at[wid, pl.ds(sec * CPS, CPS)], dst_sec)
            pltpu.async_copy(tt_hbm.at[gidx_sec.at[0]], rows0, sem0)

            def body(i, _):
                def step(rcur, scur, rnxt, snxt):
                    pltpu.make_async_copy(tt_hbm.at[gidx_sec.at[i]], rcur,
                                          scur).wait()

                    @pl.when(i + 1 < CPS)
                    def _():
                        pltpu.async_copy(tt_hbm.at[gidx_sec.at[i + 1]], rnxt,
                                         snxt)

                    pltpu.sync_copy(rcur, acc.at[dst_sec.at[i]], add=True)

                @pl.when(i % 2 == 0)
                def _():
                    step(rows0, sem0, rows1, sem1)

                @pl.when(i % 2 == 1)
                def _():
                    step(rows1, sem1, rows0, sem0)

                return 0

            lax.fori_loop(0, CPS, body, 0)
            return 0

        lax.fori_loop(0, SEC, sbody, 0)
        plsc.subcore_barrier()

        @pl.when(c == 0)
        def _():
            pltpu.sync_copy(acc.at[pl.ds(row0, ROWS_PER_TILE)],
                            part_hbm.at[0, pl.ds(row0, ROWS_PER_TILE)])

        @pl.when(c == 1)
        def _():
            pltpu.sync_copy(acc.at[pl.ds(row0, ROWS_PER_TILE)],
                            part_hbm.at[1, pl.ds(row0, ROWS_PER_TILE)])

    return k(tt, gidx_c, dst_c)


def _gidx_body(src_ref, ord_ref, out_ref):
    out_ref[...] = src_ref[...] * NORD + ord_ref[...]


def _gidx(src2, ord2):
    return pl.pallas_call(
        _gidx_body,
        out_shape=jax.ShapeDtypeStruct((E // D, D), jnp.int32),
    )(src2, ord2)


def _final_body(part_ref, cnt_ref, bias_ref, out_ref):
    p = part_ref[0] + part_ref[1]
    cnt = cnt_ref[:, 0:1]
    scale = 1.0 / jnp.sqrt(jnp.maximum(cnt, 1.0))
    out_ref[...] = p * scale + bias_ref[...]


def _finalize(part, icnt, bias2d):
    blk = 1000
    return pl.pallas_call(
        _final_body,
        grid=(N // blk,),
        in_specs=[
            pl.BlockSpec((NC, blk, D), lambda i: (0, i, 0)),
            pl.BlockSpec((blk, L), lambda i: (i, 0)),
            pl.BlockSpec((1, D), lambda i: (0, 0)),
        ],
        out_specs=pl.BlockSpec((blk, D), lambda i: (i, 0)),
        out_shape=jax.ShapeDtypeStruct((N, D), jnp.float32),
    )(part, icnt, bias2d)


def kernel(feat, edge_index, edge_order, o_emb_weight, bias):
    ei = edge_index.astype(jnp.int32)
    src = ei[0]
    dst = ei[1]
    order = edge_order.astype(jnp.int32)
    ocnt, icnt = _degrees(src.reshape(NS, NCH_A, K), dst.reshape(NS, NCH_A, K))
    gidx = _gidx(src.reshape(E // D, D), order.reshape(E // D, D))
    wt = o_emb_weight.reshape(NORD, D, D).transpose(2, 0, 1).reshape(D, NORD * D)
    t = _transform(feat, ocnt, wt)
    tt = t.reshape(N * NORD, D)
    part = _gather_scatter(tt, gidx.reshape(NW, NCH_C, K),
                           dst.reshape(NW, NCH_C, K))
    return _finalize(part, icnt, bias.reshape(1, D))
